# Initial kernel scaffold; baseline (speedup 1.0000x reference)
import jax
import jax.numpy as jnp
from jax.experimental import pallas as pl


def kernel(node_attr, edge_index, edge_attr, edge_sh, W1, b1, W2, b2, bn_gamma, bn_beta):
    raise NotImplementedError("write your pallas kernel here")


# speedup vs baseline: 4.3635x; 4.3635x over previous
import functools
import math

import jax
import jax.numpy as jnp
import numpy as np
from jax import lax
from jax.experimental import pallas as pl
from jax.experimental.pallas import tpu as pltpu
from jax.experimental.pallas import tpu_sc as plsc

N = 10000
E = 160000
IN = 32
OUT = 32
DE = 16
H = 64
NWTS = IN * OUT

NCORES = 2
NSUB = 16
NWORK = NCORES * NSUB
EPW = E // NWORK
GBATCH = 125
NBATCH = EPW // GBATCH
CHUNK = 1000
NCHUNK = EPW // CHUNK
BPC = CHUNK // GBATCH
RPS = N // NSUB
CR = CHUNK // 4

EB = 3200
INV_SQRT_IN = np.float32(1.0 / math.sqrt(IN))


@functools.cache
def _sc_gather_fn():
    mesh = plsc.VectorSubcoreMesh(core_axis_name="c", subcore_axis_name="s")

    @functools.partial(
        pl.kernel,
        mesh=mesh,
        out_type=jax.ShapeDtypeStruct((E, IN), jnp.float32),
        scratch_types=[
            pltpu.VMEM((NBATCH, GBATCH), jnp.int32),
            pltpu.VMEM((2, CHUNK, IN), jnp.float32),
            pltpu.SemaphoreType.DMA,
            pltpu.SemaphoreType.DMA,
        ],
        compiler_params=pltpu.CompilerParams(use_tc_tiling_on_sc=False),
    )
    def _sc_gather(table, idx, out, idx_v, rows_v, s_g, s_st):
        cid = lax.axis_index("c")
        sid = lax.axis_index("s")
        wid = sid * NCORES + cid
        base = wid * EPW
        pltpu.sync_copy(idx.at[wid], idx_v)
        for j in range(BPC):
            pltpu.async_copy(table.at[idx_v.at[j]],
                             rows_v.at[0, pl.ds(j * GBATCH, GBATCH)], s_g)
        for c in range(NCHUNK):
            buf = c % 2
            for j in range(BPC):
                pltpu.make_async_copy(
                    table.at[idx_v.at[0]],
                    rows_v.at[buf, pl.ds(j * GBATCH, GBATCH)], s_g).wait()
            pltpu.async_copy(rows_v.at[buf],
                             out.at[pl.ds(base + c * CHUNK, CHUNK)], s_st)
            if c + 1 < NCHUNK:
                if c >= 1:
                    pltpu.make_async_copy(out.at[pl.ds(base, CHUNK)],
                                          rows_v.at[1 - buf], s_st).wait()
                for j in range(BPC):
                    pltpu.async_copy(
                        table.at[idx_v.at[(c + 1) * BPC + j]],
                        rows_v.at[1 - buf, pl.ds(j * GBATCH, GBATCH)], s_g)
        pltpu.make_async_copy(out.at[pl.ds(base, CHUNK)],
                              rows_v.at[(NCHUNK - 2) % 2], s_st).wait()
        pltpu.make_async_copy(out.at[pl.ds(base, CHUNK)],
                              rows_v.at[(NCHUNK - 1) % 2], s_st).wait()

    return _sc_gather


@functools.cache
def _sc_scatter_fn():
    mesh = plsc.VectorSubcoreMesh(core_axis_name="c", subcore_axis_name="s")

    @functools.partial(
        pl.kernel,
        mesh=mesh,
        out_type=(jax.ShapeDtypeStruct((NCORES * N, IN), jnp.float32),
                  jax.ShapeDtypeStruct((NCORES * N, 8), jnp.float32)),
        scratch_types=[
            pltpu.VMEM((NBATCH, GBATCH), jnp.int32),
            pltpu.VMEM((NBATCH, GBATCH), jnp.int32),
            pltpu.VMEM((2, CR, 32), jnp.float32),
            pltpu.VMEM((2, CR, 32), jnp.float32),
            pltpu.VMEM((2, CR, 32), jnp.float32),
            pltpu.VMEM((2, CR, 32), jnp.float32),
            pltpu.VMEM((GBATCH, 8), jnp.float32),
            pltpu.VMEM_SHARED((N, IN), jnp.float32),
            pltpu.VMEM_SHARED((N, 8), jnp.float32),
            pltpu.SemaphoreType.DMA,
            pltpu.SemaphoreType.DMA,
        ],
        compiler_params=pltpu.CompilerParams(use_tc_tiling_on_sc=False),
    )
    def _sc_scatter(vals, idx_tp, idx_cnt, z32, z8, ones, out_tp, out_cnt,
                    itp_v, icnt_v, v0, v1, v2, v3, ones_v, acc, acc_c,
                    s_ld, s_sc):
        cid = lax.axis_index("c")
        sid = lax.axis_index("s")
        wid = sid * NCORES + cid
        base_r = wid * (EPW // 4)
        vg = (v0, v1, v2, v3)
        for g in range(4):
            pltpu.async_copy(
                vals.at[pl.ds(base_r, CR), pl.ds(g * 32, 32)],
                vg[g].at[0], s_ld)
        pltpu.sync_copy(z32.at[pl.ds(sid * RPS, RPS)],
                        acc.at[pl.ds(sid * RPS, RPS)])
        pltpu.sync_copy(z8.at[pl.ds(sid * RPS, RPS)],
                        acc_c.at[pl.ds(sid * RPS, RPS)])
        pltpu.sync_copy(ones.at[:], ones_v)
        pltpu.sync_copy(idx_tp.at[wid], itp_v)
        pltpu.sync_copy(idx_cnt.at[wid], icnt_v)
        plsc.subcore_barrier()
        for c in range(NCHUNK):
            buf = c % 2
            for g in range(4):
                pltpu.make_async_copy(
                    vals.at[pl.ds(base_r, CR), pl.ds(0, 32)],
                    vg[g].at[buf], s_ld).wait()
            if c >= 1:
                for j in range(BPC):
                    pltpu.make_async_copy(
                        vals.at[pl.ds(base_r, GBATCH), pl.ds(0, 32)],
                        vg[j % 4].at[1 - buf, pl.ds(0, GBATCH)],
                        s_sc).wait()
                for j in range(BPC):
                    pltpu.make_async_copy(
                        vals.at[pl.ds(base_r, GBATCH), pl.ds(0, 8)],
                        ones_v, s_sc).wait()
            if c + 1 < NCHUNK:
                for g in range(4):
                    pltpu.async_copy(
                        vals.at[pl.ds(base_r + (c + 1) * CR, CR),
                                pl.ds(g * 32, 32)],
                        vg[g].at[1 - buf], s_ld)
            for bb in range(2):
                for g in range(4):
                    pltpu.async_copy(
                        vg[g].at[buf, pl.ds(bb * GBATCH, GBATCH)],
                        acc.at[itp_v.at[c * BPC + bb * 4 + g]], s_sc,
                        add=True)
            for j in range(BPC):
                pltpu.async_copy(ones_v,
                                 acc_c.at[icnt_v.at[c * BPC + j]], s_sc,
                                 add=True)
        last = (NCHUNK - 1) % 2
        for j in range(BPC):
            pltpu.make_async_copy(
                vals.at[pl.ds(base_r, GBATCH), pl.ds(0, 32)],
                vg[j % 4].at[last, pl.ds(0, GBATCH)], s_sc).wait()
        for j in range(BPC):
            pltpu.make_async_copy(
                vals.at[pl.ds(base_r, GBATCH), pl.ds(0, 8)],
                ones_v, s_sc).wait()
        plsc.subcore_barrier()
        pltpu.sync_copy(acc.at[pl.ds(sid * RPS, RPS)],
                        out_tp.at[pl.ds(cid * N + sid * RPS, RPS)])
        pltpu.sync_copy(acc_c.at[pl.ds(sid * RPS, RPS)],
                        out_cnt.at[pl.ds(cid * N + sid * RPS, RPS)])

    return _sc_scatter


def _tc_fused_body(ea8, xd4, sh4, w1bd, b1bd, w2bd, rselbd, b2mbd, sel4,
                   out_ref):
    h8 = jnp.maximum(
        jnp.dot(ea8[...], w1bd[...], preferred_element_type=jnp.float32)
        + b1bd[...], 0.0)
    h4 = h8.reshape(EB // 4, 256)
    dense4 = jnp.dot(h4.astype(jnp.bfloat16), w2bd[...],
                     preferred_element_type=jnp.float32)
    tpin4 = xd4[...] * jnp.dot(sh4[...], sel4[...],
                               preferred_element_type=jnp.float32)
    bc4 = jnp.dot(tpin4.astype(jnp.bfloat16), rselbd[...],
                  preferred_element_type=jnp.float32)
    prod = dense4 * bc4
    w = 1024
    r = prod
    while w > 64:
        half = w // 2
        r = jnp.concatenate(
            [r[:, g * w: g * w + half] + r[:, g * w + half: (g + 1) * w]
             for g in range(4)], axis=1)
        w = half
    tp4 = jnp.concatenate(
        [r[:, g * 64: g * 64 + 32] + r[:, g * 64 + 32: (g + 1) * 64]
         for g in range(4)], axis=1) * INV_SQRT_IN \
        + jnp.dot(tpin4, b2mbd[...], preferred_element_type=jnp.float32)
    out_ref[...] = tp4


def _tc_fused(ea8, xd4, sh4, w1bd, b1bd, w2bd, rselbd, b2mbd, sel4):
    return pl.pallas_call(
        _tc_fused_body,
        grid=(E // EB,),
        in_specs=[
            pl.BlockSpec((EB // 8, 128), lambda i: (i, 0)),
            pl.BlockSpec((EB // 4, 128), lambda i: (i, 0)),
            pl.BlockSpec((EB // 4, 4), lambda i: (i, 0)),
            pl.BlockSpec((128, 8 * H), lambda i: (0, 0)),
            pl.BlockSpec((1, 8 * H), lambda i: (0, 0)),
            pl.BlockSpec((4 * H, 4 * NWTS), lambda i: (0, 0)),
            pl.BlockSpec((128, 4 * NWTS), lambda i: (0, 0)),
            pl.BlockSpec((128, 128), lambda i: (0, 0)),
            pl.BlockSpec((4, 128), lambda i: (0, 0)),
        ],
        out_specs=pl.BlockSpec((EB // 4, 128), lambda i: (i, 0)),
        out_shape=jax.ShapeDtypeStruct((E // 4, 128), jnp.float32),
    )(ea8, xd4, sh4, w1bd, b1bd, w2bd, rselbd, b2mbd, sel4)


def _tc_bn_body(p_ref, c_ref, na_ref, g_ref, b_ref, out_ref):
    num = p_ref[:N, :] + p_ref[N:, :]
    cnt = c_ref[:N, :1] + c_ref[N:, :1]
    o = num / jnp.maximum(cnt, 1.0) + na_ref[...]
    mean = jnp.mean(o, axis=0, keepdims=True)
    var = jnp.mean((o - mean) * (o - mean), axis=0, keepdims=True)
    out_ref[...] = (o - mean) / jnp.sqrt(var + 1e-5) * g_ref[...] + b_ref[...]


def _tc_bn(partials, counts, node_attr, gamma, beta):
    return pl.pallas_call(
        _tc_bn_body,
        out_shape=jax.ShapeDtypeStruct((N, OUT), jnp.float32),
    )(partials, counts, node_attr, gamma, beta)


def kernel(node_attr, edge_index, edge_attr, edge_sh, W1, b1, W2, b2,
           bn_gamma, bn_beta):
    f32, bf16 = jnp.float32, jnp.bfloat16
    edge_src = edge_index[0]
    edge_dst = edge_index[1]
    idx_dst = edge_dst.reshape(NWORK, NBATCH, GBATCH)
    idx_cnt = edge_src.reshape(NWORK, NBATCH, GBATCH)
    idx_tp = edge_src.reshape(NWORK, NCHUNK * 2, GBATCH, 4) \
        .transpose(0, 1, 3, 2).reshape(NWORK, NBATCH, GBATCH)

    eye8 = jnp.eye(8, dtype=f32)
    eye4 = jnp.eye(4, dtype=f32)
    w1bd = jnp.kron(eye8, W1)
    b1bd = jnp.tile(b1, (8,)).reshape(1, 8 * H)
    w2bd = jnp.kron(eye4, W2).astype(bf16)
    col = jax.lax.broadcasted_iota(jnp.int32, (IN, NWTS), 1)
    row = jax.lax.broadcasted_iota(jnp.int32, (IN, NWTS), 0)
    rsel = jnp.where(col // OUT == row, 1.0, 0.0)
    rselbd = jnp.kron(eye4, rsel).astype(bf16)
    b2mbd = jnp.kron(eye4, b2.reshape(IN, OUT) * INV_SQRT_IN)
    c2 = jax.lax.broadcasted_iota(jnp.int32, (4, 128), 1)
    r2 = jax.lax.broadcasted_iota(jnp.int32, (4, 128), 0)
    sel4 = jnp.where(c2 // 32 == r2, 1.0, 0.0).astype(f32)

    ea8 = edge_attr.reshape(E // 8, 128)
    sh4 = edge_sh.reshape(E // 4, 4)
    z32 = jnp.zeros((N, IN), f32)
    z8 = jnp.zeros((N, 8), f32)
    ones = jnp.zeros((GBATCH, 8), f32).at[:, 0].set(1.0)

    x_dst = _sc_gather_fn()(node_attr, idx_dst)
    xd4 = x_dst.reshape(E // 4, 128)
    tp4 = _tc_fused(ea8, xd4, sh4, w1bd, b1bd, w2bd, rselbd, b2mbd, sel4)
    out_tp, out_cnt = _sc_scatter_fn()(tp4, idx_tp, idx_cnt, z32, z8, ones)
    return _tc_bn(out_tp, out_cnt, node_attr,
                  bn_gamma.reshape(1, OUT), bn_beta.reshape(1, OUT))
